# baseline (speedup 1.0000x reference)
import functools

import jax
import jax.numpy as jnp
from jax import lax
from jax.experimental import pallas as pl
from jax.experimental.pallas import tpu as pltpu
from jax.experimental.pallas import tpu_sc as plsc

N = 65536
M = 16384
B = 4
NS = 32
C = 16
NB = N // B
QB = M // B
NWORK = 32
WPB = NWORK // B
QW = QB // WPB
NCHUNK = NB // 16
SEGC = 32
UNR = 1
NSEG = NCHUNK // SEGC
GQ = 16
BCAP = 48
BUFW = 80
OROW = (3 + C) * NS


def _ball_query_group(xs, ys, zs, qx, qy, qz, qr, qord, featp):
    mesh = plsc.VectorSubcoreMesh(core_axis_name="c", subcore_axis_name="s")

    @functools.partial(
        pl.kernel,
        mesh=mesh,
        out_type=[
            jax.ShapeDtypeStruct((M * OROW,), jnp.float32),
            jax.ShapeDtypeStruct((M * NS,), jnp.int32),
        ],
        compiler_params=pltpu.CompilerParams(needs_layout_passes=False),
        scratch_types=[
            pltpu.VMEM((NB,), jnp.float32),
            pltpu.VMEM((NB,), jnp.float32),
            pltpu.VMEM((NB,), jnp.float32),
            pltpu.VMEM((QW + 16,), jnp.float32),
            pltpu.VMEM((QW + 16,), jnp.float32),
            pltpu.VMEM((QW + 16,), jnp.float32),
            pltpu.VMEM((QW + 16,), jnp.float32),
            pltpu.VMEM((QW + 16,), jnp.int32),
            pltpu.VMEM((GQ, BUFW), jnp.int32),
            pltpu.VMEM((NS,), jnp.int32),
            pltpu.VMEM((NS,), jnp.int32),
            pltpu.VMEM((NS, 128), jnp.float32),
            pltpu.VMEM((NS, 128), jnp.float32),
            pltpu.VMEM((OROW,), jnp.float32),
            pltpu.VMEM((OROW,), jnp.float32),
            pltpu.SemaphoreType.DMA,
            pltpu.SemaphoreType.DMA,
            pltpu.SemaphoreType.DMA,
            pltpu.SemaphoreType.DMA,
            pltpu.VMEM((QW * NS,), jnp.int32),
        ],
    )
    def k(xs_h, ys_h, zs_h, qx_h, qy_h, qz_h, qr_h, qo_h, featp_h, outf_h,
          outi_h, pxs, pys, pzs, qxv, qyv, qzv, qrv, qov, bufs, gidxA, gidxB,
          frowsA, frowsB, orowA, orowB, gsemA, gsemB, osemA, osemB, istg):
        gidxs = (gidxA, gidxB)
        frowss = (frowsA, frowsB)
        orows = (orowA, orowB)
        gsems = (gsemA, gsemB)
        osems = (osemA, osemB)
        wid = lax.axis_index("s") * 2 + lax.axis_index("c")
        b = wid // WPB
        pbase = b * NB
        qbase = b * QB + (wid % WPB) * QW

        pltpu.sync_copy(xs_h.at[pl.ds(pbase, NB)], pxs)
        pltpu.sync_copy(ys_h.at[pl.ds(pbase, NB)], pys)
        pltpu.sync_copy(zs_h.at[pl.ds(pbase, NB)], pzs)
        pltpu.sync_copy(qx_h.at[pl.ds(qbase, QW)], qxv.at[pl.ds(0, QW)])
        pltpu.sync_copy(qy_h.at[pl.ds(qbase, QW)], qyv.at[pl.ds(0, QW)])
        pltpu.sync_copy(qz_h.at[pl.ds(qbase, QW)], qzv.at[pl.ds(0, QW)])
        pltpu.sync_copy(qr_h.at[pl.ds(qbase, QW)], qrv.at[pl.ds(0, QW)])
        pltpu.sync_copy(qo_h.at[pl.ds(qbase, QW)], qov.at[pl.ds(0, QW)])

        iota = lax.broadcasted_iota(jnp.int32, (16,), 0)

        def per_group(g, carry):
            q0 = g * GQ
            qog = qov[pl.ds(q0, 16)]
            qxg = plsc.load_gather(qxv, [qog])
            qyg = plsc.load_gather(qyv, [qog])
            qzg = plsc.load_gather(qzv, [qog])
            qrg = plsc.load_gather(qrv, [qog])
            qr2g = qrg * qrg
            qxb = [jnp.full((16,), qxg[j], jnp.float32) for j in range(GQ)]
            qyb = [jnp.full((16,), qyg[j], jnp.float32) for j in range(GQ)]
            qzb = [jnp.full((16,), qzg[j], jnp.float32) for j in range(GQ)]
            r2b = [jnp.full((16,), qr2g[j], jnp.float32) for j in range(GQ)]

            def chunk(t, cnts):
                base = t * 16
                px = pxs[pl.ds(base, 16)]
                py = pys[pl.ds(base, 16)]
                pz = pzs[pl.ds(base, 16)]
                iv = iota + jnp.full((16,), base, jnp.int32)
                out = []
                for j in range(GQ):
                    dx = px - qxb[j]
                    dy = py - qyb[j]
                    dz = pz - qzb[j]
                    d2 = dx * dx + dy * dy + dz * dz
                    m = d2 <= r2b[j]
                    plsc.store_compressed(bufs.at[j, pl.ds(cnts[j], 16)],
                                          iv, mask=m)
                    c2 = cnts[j] + plsc.all_reduce_population_count(m)[0]
                    out.append(jnp.minimum(c2, BCAP))
                return tuple(out)

            def chunkU(u, cnts):
                t = u * UNR
                for j in range(UNR):
                    cnts = chunk(t + j, cnts)
                return cnts

            def seg(s, cnts):
                mn = cnts[0]
                for j in range(1, GQ):
                    mn = jnp.minimum(mn, cnts[j])
                return lax.cond(
                    mn < NS,
                    lambda cs: lax.fori_loop(
                        s * (SEGC // UNR), (s + 1) * (SEGC // UNR), chunkU, cs),
                    lambda cs: cs,
                    cnts,
                )

            cnts = lax.fori_loop(0, NSEG, seg,
                                 tuple(jnp.int32(0) for _ in range(GQ)))

            zf = jnp.zeros((16,), jnp.float32)
            pb = jnp.full((16,), pbase, jnp.int32)

            def finalize(j, par):
                q = qog[j]
                cnt = cnts[j]
                i0 = bufs[j, pl.ds(0, 16)]
                i1 = bufs[j, pl.ds(16, 16)]
                first = i0[0]
                firstb = jnp.full((16,), first, jnp.int32)
                cntb = jnp.full((16,), cnt, jnp.int32)
                emptyb = cntb == 0
                v0 = jnp.where(iota < cntb, i0, firstb)
                v1 = jnp.where(iota + 16 < cntb, i1, firstb)
                v0 = jnp.where(emptyb, 0, v0)
                v1 = jnp.where(emptyb, 0, v1)
                g0 = jnp.where(emptyb, 0, v0 + pb)
                g1 = jnp.where(emptyb, 0, v1 + pb)
                istg[pl.ds(q * NS, 16)] = g0
                istg[pl.ds(q * NS + 16, 16)] = g1
                gidxs[par][pl.ds(0, 16)] = lax.shift_right_logical(g0, 3)
                gidxs[par][pl.ds(16, 16)] = lax.shift_right_logical(g1, 3)
                gh = pltpu.async_copy(featp_h.at[gidxs[par]],
                                      frowss[par], gsems[par])
                return (q, g0, g1, v0, v1, emptyb, j, gh)

            def process(st, par):
                q, g0, g1, v0, v1, emptyb, j, gh = st
                for ch, (arr, qb_) in enumerate(
                        ((pxs, qxb[j]), (pys, qyb[j]), (pzs, qzb[j]))):
                    c0 = plsc.load_gather(arr, [v0]) - qb_
                    c1 = plsc.load_gather(arr, [v1]) - qb_
                    orows[par][pl.ds(ch * NS, 16)] = jnp.where(emptyb, zf, c0)
                    orows[par][pl.ds(ch * NS + 16, 16)] = jnp.where(emptyb, zf, c1)
                gh.wait()
                col0 = (g0 & 7) * C
                col1 = (g1 & 7) * C
                for ch in range(C):
                    t0 = plsc.load_gather(frowss[par], [iota, col0 + ch])
                    t1 = plsc.load_gather(frowss[par], [iota + 16, col1 + ch])
                    orows[par][pl.ds((3 + ch) * NS, 16)] = jnp.where(emptyb, zf, t0)
                    orows[par][pl.ds((3 + ch) * NS + 16, 16)] = jnp.where(
                        emptyb, zf, t1)
                return pltpu.async_copy(
                    orows[par], outf_h.at[pl.ds((qbase + q) * OROW, OROW)],
                    osems[par])

            out_handles = [None, None]
            prev_state = None
            for j in range(GQ):
                par = j & 1
                st = finalize(j, par)
                if prev_state is not None:
                    pp = 1 - par
                    if out_handles[pp] is not None:
                        out_handles[pp].wait()
                    out_handles[pp] = process(prev_state, pp)
                prev_state = st
            lp = (GQ - 1) & 1
            if out_handles[lp] is not None:
                out_handles[lp].wait()
            out_handles[lp] = process(prev_state, lp)
            for pd in (0, 1):
                if out_handles[pd] is not None:
                    out_handles[pd].wait()
            return carry

        lax.fori_loop(0, QW // GQ, per_group, jnp.int32(0))
        pltpu.sync_copy(istg, outi_h.at[pl.ds(qbase * NS, QW * NS)])

    return k(xs, ys, zs, qx, qy, qz, qr, qord, featp)


def kernel(xyz, xyz_batch_cnt, new_xyz, new_xyz_r, new_xyz_batch_cnt, features):
    del xyz_batch_cnt, new_xyz_batch_cnt
    xs = xyz[:, 0]
    ys = xyz[:, 1]
    zs = xyz[:, 2]
    qx = new_xyz[:, 0]
    qy = new_xyz[:, 1]
    qz = new_xyz[:, 2]
    qr = new_xyz_r[:, 0]
    featp = features.reshape(N // 8, 8 * C)
    qord = jnp.argsort(qr.reshape(NWORK, QW), axis=1).astype(jnp.int32).reshape(-1)
    outf, outi = _ball_query_group(xs, ys, zs, qx, qy, qz, qr, qord, featp)
    new_features = outf.reshape(M, 3 + C, NS)
    idx = outi.reshape(M, NS)
    return new_features, idx


# speedup vs baseline: 203.0362x; 203.0362x over previous
import functools

import jax
import jax.numpy as jnp
from jax import lax
from jax.experimental import pallas as pl
from jax.experimental.pallas import tpu as pltpu
from jax.experimental.pallas import tpu_sc as plsc

N = 65536
M = 16384
B = 4
NS = 32
C = 16
NB = N // B
QB = M // B
NWORK = 32
WPB = NWORK // B
QW = QB // WPB
NCHUNK = NB // 16
SEGC = 32
UNR = 2
NSEG = NCHUNK // SEGC
GQ = 16
BCAP = 48
BUFW = 80
OROW = (3 + C) * NS


def _ball_query_group(xs, ys, zs, qx, qy, qz, qr, qord, featp):
    mesh = plsc.VectorSubcoreMesh(core_axis_name="c", subcore_axis_name="s")

    @functools.partial(
        pl.kernel,
        mesh=mesh,
        out_type=[
            jax.ShapeDtypeStruct((M * OROW,), jnp.float32),
            jax.ShapeDtypeStruct((M * NS,), jnp.int32),
        ],
        compiler_params=pltpu.CompilerParams(needs_layout_passes=False),
        scratch_types=[
            pltpu.VMEM((NB,), jnp.float32),
            pltpu.VMEM((NB,), jnp.float32),
            pltpu.VMEM((NB,), jnp.float32),
            pltpu.VMEM((QW + 16,), jnp.float32),
            pltpu.VMEM((QW + 16,), jnp.float32),
            pltpu.VMEM((QW + 16,), jnp.float32),
            pltpu.VMEM((QW + 16,), jnp.float32),
            pltpu.VMEM((QW + 16,), jnp.int32),
            pltpu.VMEM((GQ, BUFW), jnp.int32),
            pltpu.VMEM((NS,), jnp.int32),
            pltpu.VMEM((NS,), jnp.int32),
            pltpu.VMEM((NS, 128), jnp.float32),
            pltpu.VMEM((NS, 128), jnp.float32),
            pltpu.VMEM((OROW,), jnp.float32),
            pltpu.VMEM((OROW,), jnp.float32),
            pltpu.SemaphoreType.DMA,
            pltpu.SemaphoreType.DMA,
            pltpu.SemaphoreType.DMA,
            pltpu.SemaphoreType.DMA,
            pltpu.VMEM((QW * NS,), jnp.int32),
        ],
    )
    def k(xs_h, ys_h, zs_h, qx_h, qy_h, qz_h, qr_h, qo_h, featp_h, outf_h,
          outi_h, pxs, pys, pzs, qxv, qyv, qzv, qrv, qov, bufs, gidxA, gidxB,
          frowsA, frowsB, orowA, orowB, gsemA, gsemB, osemA, osemB, istg):
        gidxs = (gidxA, gidxB)
        frowss = (frowsA, frowsB)
        orows = (orowA, orowB)
        gsems = (gsemA, gsemB)
        osems = (osemA, osemB)
        wid = lax.axis_index("s") * 2 + lax.axis_index("c")
        b = wid // WPB
        pbase = b * NB
        qbase = b * QB + (wid % WPB) * QW

        pltpu.sync_copy(xs_h.at[pl.ds(pbase, NB)], pxs)
        pltpu.sync_copy(ys_h.at[pl.ds(pbase, NB)], pys)
        pltpu.sync_copy(zs_h.at[pl.ds(pbase, NB)], pzs)
        pltpu.sync_copy(qx_h.at[pl.ds(qbase, QW)], qxv.at[pl.ds(0, QW)])
        pltpu.sync_copy(qy_h.at[pl.ds(qbase, QW)], qyv.at[pl.ds(0, QW)])
        pltpu.sync_copy(qz_h.at[pl.ds(qbase, QW)], qzv.at[pl.ds(0, QW)])
        pltpu.sync_copy(qr_h.at[pl.ds(qbase, QW)], qrv.at[pl.ds(0, QW)])
        pltpu.sync_copy(qo_h.at[pl.ds(qbase, QW)], qov.at[pl.ds(0, QW)])

        iota = lax.broadcasted_iota(jnp.int32, (16,), 0)

        def per_group(g, carry):
            q0 = g * GQ
            qog = qov[pl.ds(q0, 16)]
            qxg = plsc.load_gather(qxv, [qog])
            qyg = plsc.load_gather(qyv, [qog])
            qzg = plsc.load_gather(qzv, [qog])
            qrg = plsc.load_gather(qrv, [qog])
            qr2g = qrg * qrg
            qxb = [jnp.full((16,), qxg[j], jnp.float32) for j in range(GQ)]
            qyb = [jnp.full((16,), qyg[j], jnp.float32) for j in range(GQ)]
            qzb = [jnp.full((16,), qzg[j], jnp.float32) for j in range(GQ)]
            r2b = [jnp.full((16,), qr2g[j], jnp.float32) for j in range(GQ)]

            def chunk(t, cnts):
                base = t * 16
                px = pxs[pl.ds(base, 16)]
                py = pys[pl.ds(base, 16)]
                pz = pzs[pl.ds(base, 16)]
                iv = iota + jnp.full((16,), base, jnp.int32)
                out = []
                for j in range(GQ):
                    dx = px - qxb[j]
                    dy = py - qyb[j]
                    dz = pz - qzb[j]
                    d2 = dx * dx + dy * dy + dz * dz
                    m = d2 <= r2b[j]
                    plsc.store_compressed(bufs.at[j, pl.ds(cnts[j], 16)],
                                          iv, mask=m)
                    c2 = cnts[j] + plsc.all_reduce_population_count(m)[0]
                    out.append(jnp.minimum(c2, BCAP))
                return tuple(out)

            def chunkU(u, cnts):
                t = u * UNR
                for j in range(UNR):
                    cnts = chunk(t + j, cnts)
                return cnts

            def seg(s, cnts):
                mn = cnts[0]
                for j in range(1, GQ):
                    mn = jnp.minimum(mn, cnts[j])
                return lax.cond(
                    mn < NS,
                    lambda cs: lax.fori_loop(
                        s * (SEGC // UNR), (s + 1) * (SEGC // UNR), chunkU, cs),
                    lambda cs: cs,
                    cnts,
                )

            cnts = lax.fori_loop(0, NSEG, seg,
                                 tuple(jnp.int32(0) for _ in range(GQ)))

            zf = jnp.zeros((16,), jnp.float32)
            pb = jnp.full((16,), pbase, jnp.int32)

            def finalize(j, par):
                q = qog[j]
                cnt = cnts[j]
                i0 = bufs[j, pl.ds(0, 16)]
                i1 = bufs[j, pl.ds(16, 16)]
                first = i0[0]
                firstb = jnp.full((16,), first, jnp.int32)
                cntb = jnp.full((16,), cnt, jnp.int32)
                emptyb = cntb == 0
                v0 = jnp.where(iota < cntb, i0, firstb)
                v1 = jnp.where(iota + 16 < cntb, i1, firstb)
                v0 = jnp.where(emptyb, 0, v0)
                v1 = jnp.where(emptyb, 0, v1)
                g0 = jnp.where(emptyb, 0, v0 + pb)
                g1 = jnp.where(emptyb, 0, v1 + pb)
                istg[pl.ds(q * NS, 16)] = g0
                istg[pl.ds(q * NS + 16, 16)] = g1
                gidxs[par][pl.ds(0, 16)] = lax.shift_right_logical(g0, 3)
                gidxs[par][pl.ds(16, 16)] = lax.shift_right_logical(g1, 3)
                gh = pltpu.async_copy(featp_h.at[gidxs[par]],
                                      frowss[par], gsems[par])
                return (q, g0, g1, v0, v1, emptyb, j, gh)

            def process(st, par):
                q, g0, g1, v0, v1, emptyb, j, gh = st
                for ch, (arr, qb_) in enumerate(
                        ((pxs, qxb[j]), (pys, qyb[j]), (pzs, qzb[j]))):
                    c0 = plsc.load_gather(arr, [v0]) - qb_
                    c1 = plsc.load_gather(arr, [v1]) - qb_
                    orows[par][pl.ds(ch * NS, 16)] = jnp.where(emptyb, zf, c0)
                    orows[par][pl.ds(ch * NS + 16, 16)] = jnp.where(emptyb, zf, c1)
                gh.wait()
                col0 = (g0 & 7) * C
                col1 = (g1 & 7) * C
                for ch in range(C):
                    t0 = plsc.load_gather(frowss[par], [iota, col0 + ch])
                    t1 = plsc.load_gather(frowss[par], [iota + 16, col1 + ch])
                    orows[par][pl.ds((3 + ch) * NS, 16)] = jnp.where(emptyb, zf, t0)
                    orows[par][pl.ds((3 + ch) * NS + 16, 16)] = jnp.where(
                        emptyb, zf, t1)
                return pltpu.async_copy(
                    orows[par], outf_h.at[pl.ds((qbase + q) * OROW, OROW)],
                    osems[par])

            out_handles = [None, None]
            prev_state = None
            for j in range(GQ):
                par = j & 1
                st = finalize(j, par)
                if prev_state is not None:
                    pp = 1 - par
                    if out_handles[pp] is not None:
                        out_handles[pp].wait()
                    out_handles[pp] = process(prev_state, pp)
                prev_state = st
            lp = (GQ - 1) & 1
            if out_handles[lp] is not None:
                out_handles[lp].wait()
            out_handles[lp] = process(prev_state, lp)
            for pd in (0, 1):
                if out_handles[pd] is not None:
                    out_handles[pd].wait()
            return carry

        lax.fori_loop(0, QW // GQ, per_group, jnp.int32(0))
        pltpu.sync_copy(istg, outi_h.at[pl.ds(qbase * NS, QW * NS)])

    return k(xs, ys, zs, qx, qy, qz, qr, qord, featp)


def kernel(xyz, xyz_batch_cnt, new_xyz, new_xyz_r, new_xyz_batch_cnt, features):
    del xyz_batch_cnt, new_xyz_batch_cnt
    xs = xyz[:, 0]
    ys = xyz[:, 1]
    zs = xyz[:, 2]
    qx = new_xyz[:, 0]
    qy = new_xyz[:, 1]
    qz = new_xyz[:, 2]
    qr = new_xyz_r[:, 0]
    featp = features.reshape(N // 8, 8 * C)
    qord = jnp.argsort(qr.reshape(NWORK, QW), axis=1).astype(jnp.int32).reshape(-1)
    outf, outi = _ball_query_group(xs, ys, zs, qx, qy, qz, qr, qord, featp)
    new_features = outf.reshape(M, 3 + C, NS)
    idx = outi.reshape(M, NS)
    return new_features, idx
